# baseline (speedup 1.0000x reference)
import jax
import jax.numpy as jnp
from jax import lax
from jax.experimental import pallas as pl
from jax.experimental.pallas import tpu as pltpu
from jax.experimental.pallas import tpu_sc as plsc

N = 10000
E = 320000
IN_FEATS = 128
HIDDEN = 256
OUT_MID = 128
ALPHA = 0.1
K = 10

NC = 2
NS = 16
L = 16

NPAD = 10240
SLICE = NPAD // NS
LAST = N - (NS - 1) * SLICE
EPT = E // NS
CHUNK = 128
NCH = (EPT + CHUNK - 1) // CHUNK
EPAD = NCH * CHUNK
REM = EPT - (NCH - 1) * CHUNK
ALOAD = (NCH + 1) * CHUNK
PAD_NODE = NPAD - 1


def _mlp_body(x_ref, w0_ref, b0_ref, w1_ref, b1_ref, w2t_ref, out_ref):
    h = jnp.dot(x_ref[...], w0_ref[...], preferred_element_type=jnp.float32)
    h = jnp.maximum(h + b0_ref[...], 0.0)
    h = jnp.dot(h, w1_ref[...], preferred_element_type=jnp.float32) + b1_ref[...]
    out_ref[...] = jnp.sum(h * w2t_ref[...], axis=1, keepdims=True)


def _mlp_project(features, W0, b0, W1, b1, W2):
    blk = 1000
    return pl.pallas_call(
        _mlp_body,
        grid=(N // blk,),
        in_specs=[
            pl.BlockSpec((blk, IN_FEATS), lambda i: (i, 0)),
            pl.BlockSpec((IN_FEATS, HIDDEN), lambda i: (0, 0)),
            pl.BlockSpec((1, HIDDEN), lambda i: (0, 0)),
            pl.BlockSpec((HIDDEN, OUT_MID), lambda i: (0, 0)),
            pl.BlockSpec((1, OUT_MID), lambda i: (0, 0)),
            pl.BlockSpec((1, OUT_MID), lambda i: (0, 0)),
        ],
        out_specs=pl.BlockSpec((blk, 1), lambda i: (i, 0)),
        out_shape=jax.ShapeDtypeStruct((N, 1), jnp.float32),
    )(features, W0, b0.reshape(1, HIDDEN), W1, b1.reshape(1, OUT_MID),
      W2.reshape(1, OUT_MID))


def _rsqrt16(x):
    y = jnp.ones((L,), jnp.float32)
    for _ in range(10):
        y = jnp.where(x * y * y > 1.2, 0.5 * y, y)
    for _ in range(7):
        y = y * (1.5 - 0.5 * x * y * y)
    return y


def _sc_body(s0_hbm, edge_hbm, b2_hbm, out_hbm,
             t_sh, agg_sh, ev_v, dst_v, msgs_v,
             ones_v, zero_v, s0_v, norm_v, s_v, a_v, b2_v, t_loc, t2_v,
             sem_g, sem_s):
    cid = lax.axis_index("c")
    sid = lax.axis_index("s")
    base = sid * SLICE
    nsl = pl.ds(base, SLICE)
    ebase = sid * EPT
    abase = jnp.minimum((ebase // CHUNK) * CHUNK, E - ALOAD)
    off = ebase - abase

    @pl.when(sid < NS - 1)
    def _():
        pltpu.async_copy(s0_hbm.at[nsl], s0_v, sem_g)

    @pl.when(sid == NS - 1)
    def _():
        pltpu.async_copy(s0_hbm.at[pl.ds(base, LAST)],
                         s0_v.at[pl.ds(0, LAST)], sem_g)

    pltpu.async_copy(edge_hbm.at[:, pl.ds(abase, ALOAD)], ev_v, sem_s)
    pltpu.sync_copy(b2_hbm, b2_v)

    ones = jnp.ones((L,), jnp.float32)
    zeros = jnp.zeros((L,), jnp.float32)
    padv = jnp.full((L,), PAD_NODE, jnp.int32)
    for g in range(CHUNK // L):
        ones_v[pl.ds(g * L, L)] = ones
    for g in range(SLICE // L):
        zero_v[pl.ds(g * L, L)] = zeros
    for g in range(REM // L, CHUNK // L):
        msgs_v[NCH - 1, pl.ds(g * L, L)] = zeros

    pltpu.make_async_copy(edge_hbm.at[:, pl.ds(0, ALOAD)], ev_v, sem_s).wait()

    def _dst_pack(j, carry):
        for g in range(CHUNK // L):
            dst_v[j, pl.ds(g * L, L)] = ev_v[1, pl.ds(off + j * CHUNK + g * L, L)]
        return carry

    lax.fori_loop(0, NCH - 1, _dst_pack, 0)
    for g in range(REM // L):
        dst_v[NCH - 1, pl.ds(g * L, L)] = ev_v[
            1, pl.ds(off + (NCH - 1) * CHUNK + g * L, L)]
    for g in range(REM // L, CHUNK // L):
        dst_v[NCH - 1, pl.ds(g * L, L)] = padv

    @pl.when(sid < NS - 1)
    def _():
        pltpu.make_async_copy(s0_hbm.at[nsl], s0_v, sem_g).wait()

    @pl.when(sid == NS - 1)
    def _():
        pltpu.make_async_copy(s0_hbm.at[pl.ds(0, LAST)],
                              s0_v.at[pl.ds(0, LAST)], sem_g).wait()
        for g in range(LAST // L, SLICE // L):
            s0_v[pl.ds(g * L, L)] = zeros

    pltpu.sync_copy(zero_v, agg_sh.at[nsl])
    plsc.subcore_barrier()

    def _deg_job(j, carry):
        pltpu.async_copy(ones_v, agg_sh.at[dst_v.at[j]], sem_s, add=True)
        return carry

    lax.fori_loop(0, NCH, _deg_job, 0)

    def _deg_drain(j, carry):
        pltpu.make_async_copy(ones_v, agg_sh.at[dst_v.at[0]], sem_s).wait()
        return carry

    lax.fori_loop(0, NCH, _deg_drain, 0)
    plsc.subcore_barrier()

    pltpu.sync_copy(agg_sh.at[nsl], a_v)

    def _norm_job(r, carry):
        for g in range(CHUNK // L):
            sl = pl.ds(r * CHUNK + g * L, L)
            d = jnp.maximum(a_v[sl], 1.0)
            y = _rsqrt16(d)
            norm_v[sl] = y
            s0v = s0_v[sl]
            s_v[sl] = s0v
            t2_v[r, pl.ds(g * L, L)] = y * s0v
        return carry

    lax.fori_loop(0, SLICE // CHUNK, _norm_job, 0)
    pltpu.sync_copy(t2_v, t_sh.at[pl.ds(sid * (SLICE // CHUNK), SLICE // CHUNK)])
    plsc.subcore_barrier()

    def _step(k, carry):
        pltpu.async_copy(t_sh, t_loc, sem_g)
        pltpu.sync_copy(zero_v, agg_sh.at[nsl])
        pltpu.make_async_copy(t_sh, t_loc, sem_g).wait()

        def _gather_job(j, c):
            for g in range(CHUNK // L):
                idx = ev_v[0, pl.ds(off + j * CHUNK + g * L, L)]
                hi = lax.shift_right_logical(idx, 7)
                lo = jnp.bitwise_and(idx, 127)
                msgs_v[j, pl.ds(g * L, L)] = plsc.load_gather(t_loc, [hi, lo])
            return c

        lax.fori_loop(0, NCH - 1, _gather_job, 0)
        for g in range(REM // L):
            idx = ev_v[0, pl.ds(off + (NCH - 1) * CHUNK + g * L, L)]
            hi = lax.shift_right_logical(idx, 7)
            lo = jnp.bitwise_and(idx, 127)
            msgs_v[NCH - 1, pl.ds(g * L, L)] = plsc.load_gather(t_loc, [hi, lo])
        plsc.subcore_barrier()

        def _scatter_fire(j, c):
            pltpu.async_copy(msgs_v.at[j], agg_sh.at[dst_v.at[j]], sem_s,
                             add=True)
            return c

        def _scatter_drain(j, c):
            pltpu.make_async_copy(msgs_v.at[0], agg_sh.at[dst_v.at[0]],
                                  sem_s).wait()
            return c

        lax.fori_loop(0, NCH, _scatter_fire, 0)
        lax.fori_loop(0, NCH, _scatter_drain, 0)
        plsc.subcore_barrier()

        pltpu.sync_copy(agg_sh.at[nsl], a_v)

        def _update_job(r, c):
            for g in range(CHUNK // L):
                sl = pl.ds(r * CHUNK + g * L, L)
                nrm = norm_v[sl]
                s_new = (1.0 - ALPHA) * (nrm * a_v[sl]) + ALPHA * s0_v[sl]
                s_v[sl] = s_new
                t2_v[r, pl.ds(g * L, L)] = nrm * s_new
            return c

        lax.fori_loop(0, SLICE // CHUNK, _update_job, 0)
        pltpu.sync_copy(t2_v, t_sh.at[pl.ds(sid * (SLICE // CHUNK), SLICE // CHUNK)])
        plsc.subcore_barrier()
        return carry

    lax.fori_loop(0, K, _step, 0)

    @pl.when(cid == 0)
    def _():
        def _out_job(g, c):
            sl = pl.ds(g * L, L)
            x = s_v[sl] + b2_v[...]
            a_v[sl] = 1.0 / (1.0 + jnp.exp(-x))
            return c

        lax.fori_loop(0, SLICE // L, _out_job, 0)

        @pl.when(sid < NS - 1)
        def _():
            pltpu.sync_copy(a_v, out_hbm.at[nsl])

        @pl.when(sid == NS - 1)
        def _():
            pltpu.sync_copy(a_v.at[pl.ds(0, LAST)],
                            out_hbm.at[pl.ds(base, LAST)])


def _diffuse(s0, edge, b2_16):
    mesh = plsc.VectorSubcoreMesh(
        core_axis_name="c", subcore_axis_name="s",
        num_cores=NC, num_subcores=NS)
    run = pl.kernel(
        _sc_body,
        out_type=jax.ShapeDtypeStruct((N,), jnp.float32),
        mesh=mesh,
        compiler_params=pltpu.CompilerParams(needs_layout_passes=False),
        scratch_types=[
            pltpu.VMEM_SHARED((NPAD // CHUNK, CHUNK), jnp.float32),
            pltpu.VMEM_SHARED((NPAD,), jnp.float32),
            pltpu.VMEM((2, ALOAD), jnp.int32),
            pltpu.VMEM((NCH, CHUNK), jnp.int32),
            pltpu.VMEM((NCH, CHUNK), jnp.float32),
            pltpu.VMEM((CHUNK,), jnp.float32),
            pltpu.VMEM((SLICE,), jnp.float32),
            pltpu.VMEM((SLICE,), jnp.float32),
            pltpu.VMEM((SLICE,), jnp.float32),
            pltpu.VMEM((SLICE,), jnp.float32),
            pltpu.VMEM((SLICE,), jnp.float32),
            pltpu.VMEM((L,), jnp.float32),
            pltpu.VMEM((NPAD // CHUNK, CHUNK), jnp.float32),
            pltpu.VMEM((SLICE // CHUNK, CHUNK), jnp.float32),
            pltpu.SemaphoreType.DMA,
            pltpu.SemaphoreType.DMA,
        ],
    )
    return run(s0, edge, b2_16)


def kernel(features, edge_index, W0, b0, W1, b1, W2, b2):
    s0 = _mlp_project(features, W0, b0, W1, b1, W2)
    edge = edge_index.astype(jnp.int32)
    b2_16 = jnp.broadcast_to(b2.astype(jnp.float32), (L,))
    out = _diffuse(s0.reshape(N), edge, b2_16)
    return out.reshape(N, 1)


# speedup vs baseline: 1.2886x; 1.2886x over previous
import jax
import jax.numpy as jnp
from jax import lax
from jax.experimental import pallas as pl
from jax.experimental.pallas import tpu as pltpu
from jax.experimental.pallas import tpu_sc as plsc

N = 10000
E = 320000
IN_FEATS = 128
HIDDEN = 256
OUT_MID = 128
ALPHA = 0.1
K = 10

NC = 2
NS = 16
L = 16

NPAD = 10240
SLICE = NPAD // NS
LAST = N - (NS - 1) * SLICE
EPT = E // NS
CHUNK = 128
NCH = (EPT + CHUNK - 1) // CHUNK
EPAD = NCH * CHUNK
REM = EPT - (NCH - 1) * CHUNK
ALOAD = (NCH + 1) * CHUNK
PAD_NODE = NPAD - 1


def _mlp_body(x_ref, w0_ref, b0_ref, w1_ref, b1_ref, w2t_ref, out_ref):
    h = jnp.dot(x_ref[...], w0_ref[...], preferred_element_type=jnp.float32)
    h = jnp.maximum(h + b0_ref[...], 0.0)
    h = jnp.dot(h, w1_ref[...], preferred_element_type=jnp.float32) + b1_ref[...]
    out_ref[...] = jnp.sum(h * w2t_ref[...], axis=1, keepdims=True)


def _mlp_project(features, W0, b0, W1, b1, W2):
    blk = 1000
    return pl.pallas_call(
        _mlp_body,
        grid=(N // blk,),
        in_specs=[
            pl.BlockSpec((blk, IN_FEATS), lambda i: (i, 0)),
            pl.BlockSpec((IN_FEATS, HIDDEN), lambda i: (0, 0)),
            pl.BlockSpec((1, HIDDEN), lambda i: (0, 0)),
            pl.BlockSpec((HIDDEN, OUT_MID), lambda i: (0, 0)),
            pl.BlockSpec((1, OUT_MID), lambda i: (0, 0)),
            pl.BlockSpec((1, OUT_MID), lambda i: (0, 0)),
        ],
        out_specs=pl.BlockSpec((blk, 1), lambda i: (i, 0)),
        out_shape=jax.ShapeDtypeStruct((N, 1), jnp.float32),
    )(features, W0, b0.reshape(1, HIDDEN), W1, b1.reshape(1, OUT_MID),
      W2.reshape(1, OUT_MID))


def _rsqrt16(x):
    y = jnp.ones((L,), jnp.float32)
    for _ in range(10):
        y = jnp.where(x * y * y > 1.2, 0.5 * y, y)
    for _ in range(7):
        y = y * (1.5 - 0.5 * x * y * y)
    return y


def _sc_body(s0_hbm, edge_hbm, b2_hbm, out_hbm,
             t_sh, agg_sh, ev_v, src_v, dst_v, msgs_v,
             ones_v, zero_v, s0_v, norm_v, s_v, a_v, b2_v, t_loc, t2_v,
             sem_g, sem_s):
    cid = lax.axis_index("c")
    sid = lax.axis_index("s")
    base = sid * SLICE
    nsl = pl.ds(base, SLICE)
    ebase = sid * EPT
    abase = jnp.minimum((ebase // CHUNK) * CHUNK, E - ALOAD)
    off = ebase - abase

    @pl.when(sid < NS - 1)
    def _():
        pltpu.async_copy(s0_hbm.at[nsl], s0_v, sem_g)

    @pl.when(sid == NS - 1)
    def _():
        pltpu.async_copy(s0_hbm.at[pl.ds(base, LAST)],
                         s0_v.at[pl.ds(0, LAST)], sem_g)

    pltpu.async_copy(edge_hbm.at[:, pl.ds(abase, ALOAD)], ev_v, sem_s)
    pltpu.sync_copy(b2_hbm, b2_v)

    ones = jnp.ones((L,), jnp.float32)
    zeros = jnp.zeros((L,), jnp.float32)
    padv = jnp.full((L,), PAD_NODE, jnp.int32)
    for g in range(CHUNK // L):
        ones_v[pl.ds(g * L, L)] = ones
    for g in range(SLICE // L):
        zero_v[pl.ds(g * L, L)] = zeros
    for g in range(REM // L, CHUNK // L):
        msgs_v[NCH - 1, pl.ds(g * L, L)] = zeros

    pltpu.make_async_copy(edge_hbm.at[:, pl.ds(0, ALOAD)], ev_v, sem_s).wait()

    def _dst_pack(j, carry):
        for g in range(CHUNK // L):
            sl = pl.ds(off + j * CHUNK + g * L, L)
            dst_v[j, pl.ds(g * L, L)] = ev_v[1, sl]
            src_v[pl.ds(j * CHUNK + g * L, L)] = ev_v[0, sl]
        return carry

    lax.fori_loop(0, NCH - 1, _dst_pack, 0)
    for g in range(REM // L):
        sl = pl.ds(off + (NCH - 1) * CHUNK + g * L, L)
        dst_v[NCH - 1, pl.ds(g * L, L)] = ev_v[1, sl]
        src_v[pl.ds((NCH - 1) * CHUNK + g * L, L)] = ev_v[0, sl]
    for g in range(REM // L, CHUNK // L):
        dst_v[NCH - 1, pl.ds(g * L, L)] = padv
        src_v[pl.ds((NCH - 1) * CHUNK + g * L, L)] = padv

    @pl.when(sid < NS - 1)
    def _():
        pltpu.make_async_copy(s0_hbm.at[nsl], s0_v, sem_g).wait()

    @pl.when(sid == NS - 1)
    def _():
        pltpu.make_async_copy(s0_hbm.at[pl.ds(0, LAST)],
                              s0_v.at[pl.ds(0, LAST)], sem_g).wait()
        for g in range(LAST // L, SLICE // L):
            s0_v[pl.ds(g * L, L)] = zeros

    pltpu.sync_copy(zero_v, agg_sh.at[nsl])
    plsc.subcore_barrier()

    def _deg_job(j, carry):
        pltpu.async_copy(ones_v, agg_sh.at[dst_v.at[j]], sem_s, add=True)
        return carry

    lax.fori_loop(0, NCH, _deg_job, 0)

    def _deg_drain(j, carry):
        pltpu.make_async_copy(ones_v, agg_sh.at[dst_v.at[0]], sem_s).wait()
        return carry

    lax.fori_loop(0, NCH, _deg_drain, 0)
    plsc.subcore_barrier()

    pltpu.sync_copy(agg_sh.at[nsl], a_v)

    def _norm_job(r, carry):
        for g in range(CHUNK // L):
            sl = pl.ds(r * CHUNK + g * L, L)
            d = jnp.maximum(a_v[sl], 1.0)
            y = _rsqrt16(d)
            norm_v[sl] = y
            s0v = s0_v[sl]
            s_v[sl] = s0v
            t2_v[r, pl.ds(g * L, L)] = y * s0v
        return carry

    lax.fori_loop(0, SLICE // CHUNK, _norm_job, 0)
    pltpu.sync_copy(t2_v, t_sh.at[pl.ds(sid * (SLICE // CHUNK), SLICE // CHUNK)])
    plsc.subcore_barrier()

    def _step(k, carry):
        pltpu.async_copy(t_sh, t_loc, sem_g)
        pltpu.sync_copy(zero_v, agg_sh.at[nsl])
        pltpu.make_async_copy(t_sh, t_loc, sem_g).wait()

        def _gather_job(j, c):
            for g in range(CHUNK // L):
                idx = src_v[pl.ds(j * CHUNK + g * L, L)]
                hi = lax.shift_right_logical(idx, 7)
                lo = jnp.bitwise_and(idx, 127)
                msgs_v[j, pl.ds(g * L, L)] = plsc.load_gather(t_loc, [hi, lo])
            return c

        lax.fori_loop(0, NCH, _gather_job, 0)
        plsc.subcore_barrier()

        def _scatter_fire(j, c):
            pltpu.async_copy(msgs_v.at[j], agg_sh.at[dst_v.at[j]], sem_s,
                             add=True)
            return c

        def _scatter_drain(j, c):
            pltpu.make_async_copy(msgs_v.at[0], agg_sh.at[dst_v.at[0]],
                                  sem_s).wait()
            return c

        lax.fori_loop(0, NCH, _scatter_fire, 0)
        lax.fori_loop(0, NCH, _scatter_drain, 0)
        plsc.subcore_barrier()

        pltpu.sync_copy(agg_sh.at[nsl], a_v)

        def _update_job(r, c):
            for g in range(CHUNK // L):
                sl = pl.ds(r * CHUNK + g * L, L)
                nrm = norm_v[sl]
                s_new = (1.0 - ALPHA) * (nrm * a_v[sl]) + ALPHA * s0_v[sl]
                s_v[sl] = s_new
                t2_v[r, pl.ds(g * L, L)] = nrm * s_new
            return c

        lax.fori_loop(0, SLICE // CHUNK, _update_job, 0)
        pltpu.sync_copy(t2_v, t_sh.at[pl.ds(sid * (SLICE // CHUNK), SLICE // CHUNK)])
        plsc.subcore_barrier()
        return carry

    lax.fori_loop(0, K, _step, 0)

    @pl.when(cid == 0)
    def _():
        def _out_job(g, c):
            sl = pl.ds(g * L, L)
            x = s_v[sl] + b2_v[...]
            a_v[sl] = 1.0 / (1.0 + jnp.exp(-x))
            return c

        lax.fori_loop(0, SLICE // L, _out_job, 0)

        @pl.when(sid < NS - 1)
        def _():
            pltpu.sync_copy(a_v, out_hbm.at[nsl])

        @pl.when(sid == NS - 1)
        def _():
            pltpu.sync_copy(a_v.at[pl.ds(0, LAST)],
                            out_hbm.at[pl.ds(base, LAST)])


def _diffuse(s0, edge, b2_16):
    mesh = plsc.VectorSubcoreMesh(
        core_axis_name="c", subcore_axis_name="s",
        num_cores=NC, num_subcores=NS)
    run = pl.kernel(
        _sc_body,
        out_type=jax.ShapeDtypeStruct((N,), jnp.float32),
        mesh=mesh,
        compiler_params=pltpu.CompilerParams(needs_layout_passes=False),
        scratch_types=[
            pltpu.VMEM_SHARED((NPAD // CHUNK, CHUNK), jnp.float32),
            pltpu.VMEM_SHARED((NPAD,), jnp.float32),
            pltpu.VMEM((2, ALOAD), jnp.int32),
            pltpu.VMEM((EPAD,), jnp.int32),
            pltpu.VMEM((NCH, CHUNK), jnp.int32),
            pltpu.VMEM((NCH, CHUNK), jnp.float32),
            pltpu.VMEM((CHUNK,), jnp.float32),
            pltpu.VMEM((SLICE,), jnp.float32),
            pltpu.VMEM((SLICE,), jnp.float32),
            pltpu.VMEM((SLICE,), jnp.float32),
            pltpu.VMEM((SLICE,), jnp.float32),
            pltpu.VMEM((SLICE,), jnp.float32),
            pltpu.VMEM((L,), jnp.float32),
            pltpu.VMEM((NPAD // CHUNK, CHUNK), jnp.float32),
            pltpu.VMEM((SLICE // CHUNK, CHUNK), jnp.float32),
            pltpu.SemaphoreType.DMA,
            pltpu.SemaphoreType.DMA,
        ],
    )
    return run(s0, edge, b2_16)


def kernel(features, edge_index, W0, b0, W1, b1, W2, b2):
    s0 = _mlp_project(features, W0, b0, W1, b1, W2)
    edge = edge_index.astype(jnp.int32)
    b2_16 = jnp.broadcast_to(b2.astype(jnp.float32), (L,))
    out = _diffuse(s0.reshape(N), edge, b2_16)
    return out.reshape(N, 1)


# speedup vs baseline: 1.7529x; 1.3603x over previous
import jax
import jax.numpy as jnp
from jax import lax
from jax.experimental import pallas as pl
from jax.experimental.pallas import tpu as pltpu
from jax.experimental.pallas import tpu_sc as plsc

N = 10000
E = 320000
IN_FEATS = 128
HIDDEN = 256
OUT_MID = 128
ALPHA = 0.1
K = 10

NC = 2
NS = 16
L = 16

NPAD = 10240
SLICE = NPAD // NS
LAST = N - (NS - 1) * SLICE
EPT = E // NS
CHUNK = 128
NCH = (EPT + CHUNK - 1) // CHUNK
EPAD = NCH * CHUNK
REM = EPT - (NCH - 1) * CHUNK
ALOAD = (NCH + 1) * CHUNK
PAD_NODE = NPAD - 1


def _mlp_body(x_ref, w0_ref, b0_ref, w1_ref, b1_ref, w2t_ref, out_ref):
    h = jnp.dot(x_ref[...], w0_ref[...], preferred_element_type=jnp.float32)
    h = jnp.maximum(h + b0_ref[...], 0.0)
    h = jnp.dot(h, w1_ref[...], preferred_element_type=jnp.float32) + b1_ref[...]
    out_ref[...] = jnp.sum(h * w2t_ref[...], axis=1, keepdims=True)


def _mlp_project(features, W0, b0, W1, b1, W2):
    blk = 1000
    return pl.pallas_call(
        _mlp_body,
        grid=(N // blk,),
        in_specs=[
            pl.BlockSpec((blk, IN_FEATS), lambda i: (i, 0)),
            pl.BlockSpec((IN_FEATS, HIDDEN), lambda i: (0, 0)),
            pl.BlockSpec((1, HIDDEN), lambda i: (0, 0)),
            pl.BlockSpec((HIDDEN, OUT_MID), lambda i: (0, 0)),
            pl.BlockSpec((1, OUT_MID), lambda i: (0, 0)),
            pl.BlockSpec((1, OUT_MID), lambda i: (0, 0)),
        ],
        out_specs=pl.BlockSpec((blk, 1), lambda i: (i, 0)),
        out_shape=jax.ShapeDtypeStruct((N, 1), jnp.float32),
    )(features, W0, b0.reshape(1, HIDDEN), W1, b1.reshape(1, OUT_MID),
      W2.reshape(1, OUT_MID))


def _rsqrt16(x):
    y = jnp.ones((L,), jnp.float32)
    for _ in range(10):
        y = jnp.where(x * y * y > 1.2, 0.5 * y, y)
    for _ in range(7):
        y = y * (1.5 - 0.5 * x * y * y)
    return y


def _sc_body(s0_hbm, edge_hbm, b2_hbm, out_hbm,
             t_sh, agg_sh, ev_v, src_v, dst_v, msgs_v,
             ones_v, zero_v, s0_v, norm_v, s_v, a_v, b2_v, t_loc, t2_v,
             sem_g, sem_s):
    cid = lax.axis_index("c")
    sid = lax.axis_index("s")
    base = sid * SLICE
    nsl = pl.ds(base, SLICE)
    ebase = sid * EPT
    abase = jnp.minimum((ebase // CHUNK) * CHUNK, E - ALOAD)
    off = ebase - abase

    @pl.when(sid < NS - 1)
    def _():
        pltpu.async_copy(s0_hbm.at[nsl], s0_v, sem_g)

    @pl.when(sid == NS - 1)
    def _():
        pltpu.async_copy(s0_hbm.at[pl.ds(base, LAST)],
                         s0_v.at[pl.ds(0, LAST)], sem_g)

    pltpu.async_copy(edge_hbm.at[:, pl.ds(abase, ALOAD)], ev_v, sem_s)
    pltpu.sync_copy(b2_hbm, b2_v)

    ones = jnp.ones((L,), jnp.float32)
    zeros = jnp.zeros((L,), jnp.float32)
    padv = jnp.full((L,), PAD_NODE, jnp.int32)
    for g in range(CHUNK // L):
        ones_v[pl.ds(g * L, L)] = ones
    for g in range(SLICE // L):
        zero_v[pl.ds(g * L, L)] = zeros
    for g in range(REM // L, CHUNK // L):
        msgs_v[NCH - 1, pl.ds(g * L, L)] = zeros

    pltpu.make_async_copy(edge_hbm.at[:, pl.ds(0, ALOAD)], ev_v, sem_s).wait()

    def _dst_pack(j, carry):
        for g in range(CHUNK // L):
            sl = pl.ds(off + j * CHUNK + g * L, L)
            dst_v[j, pl.ds(g * L, L)] = ev_v[1, sl]
            src_v[pl.ds(j * CHUNK + g * L, L)] = ev_v[0, sl]
        return carry

    lax.fori_loop(0, NCH - 1, _dst_pack, 0)
    for g in range(REM // L):
        sl = pl.ds(off + (NCH - 1) * CHUNK + g * L, L)
        dst_v[NCH - 1, pl.ds(g * L, L)] = ev_v[1, sl]
        src_v[pl.ds((NCH - 1) * CHUNK + g * L, L)] = ev_v[0, sl]
    for g in range(REM // L, CHUNK // L):
        dst_v[NCH - 1, pl.ds(g * L, L)] = padv
        src_v[pl.ds((NCH - 1) * CHUNK + g * L, L)] = padv

    @pl.when(sid < NS - 1)
    def _():
        pltpu.make_async_copy(s0_hbm.at[nsl], s0_v, sem_g).wait()

    @pl.when(sid == NS - 1)
    def _():
        pltpu.make_async_copy(s0_hbm.at[pl.ds(0, LAST)],
                              s0_v.at[pl.ds(0, LAST)], sem_g).wait()
        for g in range(LAST // L, SLICE // L):
            s0_v[pl.ds(g * L, L)] = zeros

    pltpu.sync_copy(zero_v, agg_sh.at[nsl])
    plsc.subcore_barrier()

    def _deg_job(j, carry):
        pltpu.async_copy(ones_v, agg_sh.at[dst_v.at[j]], sem_s, add=True)
        return carry

    lax.fori_loop(0, NCH, _deg_job, 0)

    def _deg_drain(j, carry):
        pltpu.make_async_copy(ones_v, agg_sh.at[dst_v.at[0]], sem_s).wait()
        return carry

    lax.fori_loop(0, NCH, _deg_drain, 0)
    plsc.subcore_barrier()

    pltpu.sync_copy(agg_sh.at[nsl], a_v)

    def _norm_job(r, carry):
        for g in range(CHUNK // L):
            sl = pl.ds(r * CHUNK + g * L, L)
            d = jnp.maximum(a_v[sl], 1.0)
            y = _rsqrt16(d)
            norm_v[sl] = y
            s0v = s0_v[sl]
            s_v[sl] = s0v
            t2_v[r, pl.ds(g * L, L)] = y * s0v
        return carry

    lax.fori_loop(0, SLICE // CHUNK, _norm_job, 0)
    pltpu.sync_copy(t2_v, t_sh.at[pl.ds(sid * (SLICE // CHUNK), SLICE // CHUNK)])
    plsc.subcore_barrier()

    def _step(k, carry):
        pltpu.async_copy(t_sh, t_loc, sem_g)
        pltpu.sync_copy(zero_v, agg_sh.at[nsl])
        pltpu.make_async_copy(t_sh, t_loc, sem_g).wait()
        plsc.subcore_barrier()

        def _gs_job(j, c):
            for g in range(CHUNK // L):
                idx = src_v[pl.ds(j * CHUNK + g * L, L)]
                hi = lax.shift_right_logical(idx, 7)
                lo = jnp.bitwise_and(idx, 127)
                msgs_v[j, pl.ds(g * L, L)] = plsc.load_gather(t_loc, [hi, lo])
            pltpu.async_copy(msgs_v.at[j], agg_sh.at[dst_v.at[j]], sem_s,
                             add=True)
            return c

        def _scatter_drain(j, c):
            pltpu.make_async_copy(msgs_v.at[0], agg_sh.at[dst_v.at[0]],
                                  sem_s).wait()
            return c

        lax.fori_loop(0, NCH, _gs_job, 0)
        lax.fori_loop(0, NCH, _scatter_drain, 0)
        plsc.subcore_barrier()

        pltpu.sync_copy(agg_sh.at[nsl], a_v)

        def _update_job(r, c):
            for g in range(CHUNK // L):
                sl = pl.ds(r * CHUNK + g * L, L)
                nrm = norm_v[sl]
                s_new = (1.0 - ALPHA) * (nrm * a_v[sl]) + ALPHA * s0_v[sl]
                s_v[sl] = s_new
                t2_v[r, pl.ds(g * L, L)] = nrm * s_new
            return c

        lax.fori_loop(0, SLICE // CHUNK, _update_job, 0)
        pltpu.sync_copy(t2_v, t_sh.at[pl.ds(sid * (SLICE // CHUNK), SLICE // CHUNK)])
        plsc.subcore_barrier()
        return carry

    lax.fori_loop(0, K, _step, 0)

    @pl.when(cid == 0)
    def _():
        def _out_job(g, c):
            sl = pl.ds(g * L, L)
            x = s_v[sl] + b2_v[...]
            a_v[sl] = 1.0 / (1.0 + jnp.exp(-x))
            return c

        lax.fori_loop(0, SLICE // L, _out_job, 0)

        @pl.when(sid < NS - 1)
        def _():
            pltpu.sync_copy(a_v, out_hbm.at[nsl])

        @pl.when(sid == NS - 1)
        def _():
            pltpu.sync_copy(a_v.at[pl.ds(0, LAST)],
                            out_hbm.at[pl.ds(base, LAST)])


def _diffuse(s0, edge, b2_16):
    mesh = plsc.VectorSubcoreMesh(
        core_axis_name="c", subcore_axis_name="s",
        num_cores=NC, num_subcores=NS)
    run = pl.kernel(
        _sc_body,
        out_type=jax.ShapeDtypeStruct((N,), jnp.float32),
        mesh=mesh,
        compiler_params=pltpu.CompilerParams(needs_layout_passes=False),
        scratch_types=[
            pltpu.VMEM_SHARED((NPAD // CHUNK, CHUNK), jnp.float32),
            pltpu.VMEM_SHARED((NPAD,), jnp.float32),
            pltpu.VMEM((2, ALOAD), jnp.int32),
            pltpu.VMEM((EPAD,), jnp.int32),
            pltpu.VMEM((NCH, CHUNK), jnp.int32),
            pltpu.VMEM((NCH, CHUNK), jnp.float32),
            pltpu.VMEM((CHUNK,), jnp.float32),
            pltpu.VMEM((SLICE,), jnp.float32),
            pltpu.VMEM((SLICE,), jnp.float32),
            pltpu.VMEM((SLICE,), jnp.float32),
            pltpu.VMEM((SLICE,), jnp.float32),
            pltpu.VMEM((SLICE,), jnp.float32),
            pltpu.VMEM((L,), jnp.float32),
            pltpu.VMEM((NPAD // CHUNK, CHUNK), jnp.float32),
            pltpu.VMEM((SLICE // CHUNK, CHUNK), jnp.float32),
            pltpu.SemaphoreType.DMA,
            pltpu.SemaphoreType.DMA,
        ],
    )
    return run(s0, edge, b2_16)


def kernel(features, edge_index, W0, b0, W1, b1, W2, b2):
    s0 = _mlp_project(features, W0, b0, W1, b1, W2)
    edge = edge_index.astype(jnp.int32)
    b2_16 = jnp.broadcast_to(b2.astype(jnp.float32), (L,))
    out = _diffuse(s0.reshape(N), edge, b2_16)
    return out.reshape(N, 1)
